# Initial kernel scaffold; baseline (speedup 1.0000x reference)
import jax
import jax.numpy as jnp
from jax.experimental import pallas as pl


def kernel(edge_index_0, edge_index_1, W1, b1, W2, b2, W3, b3, att_W, tn_W, tn_Wb, tn_bias, fc1_W, fc1_b, fc2_W, fc2_b):
    raise NotImplementedError("write your pallas kernel here")


# speedup vs baseline: 20.6842x; 20.6842x over previous
import functools

import jax
import jax.numpy as jnp
from jax import lax
from jax.experimental import pallas as pl
from jax.experimental.pallas import tpu as pltpu
from jax.experimental.pallas import tpu_sc as plsc

N = 50_000
E = 1_600_000
G3D = 32
TN = 16
NC = 2
NS = 16
LW = 128
K = 8
RT = 784
NCHUNK = RT // K
ROWS = NS * RT
EPAD = ROWS * LW
ACC = 50_048
TPT = ACC // NS
MT = NC * ACC
OFF1 = ACC
PAD_SRC = OFF1 + N
PAD_DST = N
ZR = 136
RB = MT // 32
PB = 2000

_f32 = jnp.float32


@functools.lru_cache(maxsize=None)
def _mesh():
    return plsc.VectorSubcoreMesh(core_axis_name="c", subcore_axis_name="s",
                                  num_cores=NC, num_subcores=NS)


def _zero_1d(buf, n):
    def body(j, carry):
        buf[pl.ds(j * 16, 16)] = jnp.zeros((16,), _f32)
        return carry
    lax.fori_loop(0, n // 16, body, 0)


def _zero_2d(buf, n):
    def body(j, carry):
        buf[j, pl.ds(0, 16)] = jnp.zeros((16,), _f32)
        buf[j, pl.ds(16, 16)] = jnp.zeros((16,), _f32)
        return carry
    lax.fori_loop(0, n, body, 0)


def _deg_body(src_hbm, dst_hbm, in_deg, out_deg, idx_v, ones_v, tbuf,
              in_acc, out_acc):
    c = lax.axis_index("c")
    s = lax.axis_index("s")
    _zero_1d(tbuf, TPT)
    pltpu.sync_copy(tbuf, in_acc.at[pl.ds(s * TPT, TPT)])
    pltpu.sync_copy(tbuf, out_acc.at[pl.ds(s * TPT, TPT)])
    pltpu.sync_copy(tbuf, out_acc.at[pl.ds(ACC + s * TPT, TPT)])
    for i in range(LW // 16):
        ones_v[pl.ds(i * 16, 16)] = jnp.full((16,), 1.0, _f32)
    plsc.subcore_barrier()
    base = s * RT

    def body(i, carry):
        r0 = base + i * K
        pltpu.sync_copy(dst_hbm.at[c, pl.ds(r0, K)], idx_v)
        for j in range(K):
            pltpu.sync_copy(ones_v, in_acc.at[idx_v.at[j]], add=True)
        pltpu.sync_copy(src_hbm.at[c, pl.ds(r0, K)], idx_v)
        for j in range(K):
            pltpu.sync_copy(ones_v, out_acc.at[idx_v.at[j]], add=True)
        return carry

    lax.fori_loop(0, NCHUNK, body, 0)
    plsc.subcore_barrier()
    dst0 = c * ACC + s * TPT
    pltpu.sync_copy(in_acc.at[pl.ds(s * TPT, TPT)], tbuf)
    pltpu.sync_copy(tbuf, in_deg.at[pl.ds(dst0, TPT)])
    pltpu.sync_copy(out_acc.at[pl.ds(dst0, TPT)], tbuf)
    pltpu.sync_copy(tbuf, out_deg.at[pl.ds(dst0, TPT)])


@functools.lru_cache(maxsize=None)
def _deg_kernel():
    return pl.kernel(
        _deg_body,
        out_type=(jax.ShapeDtypeStruct((MT,), _f32),
                  jax.ShapeDtypeStruct((MT,), _f32)),
        mesh=_mesh(),
        scratch_types=[
            pltpu.VMEM((K, LW), jnp.int32),
            pltpu.VMEM((LW,), _f32),
            pltpu.VMEM((TPT,), _f32),
            pltpu.VMEM_SHARED((ACC,), _f32),
            pltpu.VMEM_SHARED((MT,), _f32),
        ],
        compiler_params=pltpu.CompilerParams(use_tc_tiling_on_sc=False),
    )


def _seg_body(D, g_hbm, src_hbm, dst_hbm, agg_out, sidx, didx, rows_v, tbuf,
              acc, sem):
    KD = K if D == 1 else K // 2
    c = lax.axis_index("c")
    s = lax.axis_index("s")
    if D == 1:
        _zero_1d(tbuf, TPT)
        pltpu.sync_copy(tbuf, acc.at[pl.ds(s * TPT, TPT)])
    else:
        _zero_2d(tbuf, ZR)
        for q in range(TPT // ZR):
            pltpu.sync_copy(tbuf, acc.at[pl.ds(s * TPT + q * ZR, ZR)])
    plsc.subcore_barrier()
    base = s * RT

    def body(i, carry):
        r0 = base + i * KD
        pltpu.sync_copy(src_hbm.at[c, pl.ds(r0, KD)], sidx)
        pltpu.sync_copy(dst_hbm.at[c, pl.ds(r0, KD)], didx)
        descs = [pltpu.async_copy(g_hbm.at[sidx.at[j]], rows_v.at[j], sem)
                 for j in range(KD)]
        for d in descs:
            d.wait()
        for j in range(KD):
            pltpu.sync_copy(rows_v.at[j], acc.at[didx.at[j]], add=True)
        return carry

    lax.fori_loop(0, RT // KD, body, 0)
    plsc.subcore_barrier()
    if D == 1:
        dst0 = c * ACC + s * TPT
        pltpu.sync_copy(acc.at[pl.ds(s * TPT, TPT)], tbuf)
        pltpu.sync_copy(tbuf, agg_out.at[pl.ds(dst0, TPT)])
    else:
        for q in range(TPT // ZR):
            r0 = s * TPT + q * ZR
            pltpu.sync_copy(acc.at[pl.ds(r0, ZR)], tbuf)
            pltpu.sync_copy(tbuf, agg_out.at[pl.ds(c * ACC + r0, ZR)])


@functools.lru_cache(maxsize=None)
def _seg_kernel(D):
    if D == 1:
        gshape, rshape, tshape, ashape = (MT,), (K, LW), (TPT,), (ACC,)
        KD = K
    else:
        KD = K // 2
        gshape, rshape, tshape, ashape = ((MT, D), (KD, LW, D), (ZR, D),
                                          (ACC, D))
    return pl.kernel(
        functools.partial(_seg_body, D),
        out_type=jax.ShapeDtypeStruct(gshape, _f32),
        mesh=_mesh(),
        scratch_types=[
            pltpu.VMEM((KD, LW), jnp.int32),
            pltpu.VMEM((KD, LW), jnp.int32),
            pltpu.VMEM(rshape, _f32),
            pltpu.VMEM(tshape, _f32),
            pltpu.VMEM_SHARED(ashape, _f32),
            pltpu.SemaphoreType.DMA,
        ],
        compiler_params=pltpu.CompilerParams(use_tc_tiling_on_sc=False),
    )


def _prep_body(ind_ref, outd_ref, inn_ref, onn_ref, g1_ref):
    ind = ind_ref[...]
    onn = lax.rsqrt(jnp.maximum(outd_ref[...], 1.0))
    inn_ref[...] = lax.rsqrt(jnp.maximum(ind, 1.0))
    onn_ref[...] = onn
    g1_ref[...] = ind * onn


_prep_call = pl.pallas_call(
    _prep_body,
    out_shape=(jax.ShapeDtypeStruct((MT // LW, LW), _f32),) * 3,
)


def _l1_body(agg_ref, inn_ref, onn_ref, w_ref, b_ref, g_ref):
    x = agg_ref[...] * inn_ref[...]
    y = x * w_ref[...] + b_ref[...]
    g_ref[...] = jnp.maximum(y, 0.0) * onn_ref[...]


_l1_call = pl.pallas_call(
    _l1_body,
    grid=(MT // RB,),
    in_specs=[
        pl.BlockSpec((RB, 1), lambda i: (i, 0)),
        pl.BlockSpec((RB, 1), lambda i: (i, 0)),
        pl.BlockSpec((RB, 1), lambda i: (i, 0)),
        pl.BlockSpec((1, G3D), lambda i: (0, 0)),
        pl.BlockSpec((1, G3D), lambda i: (0, 0)),
    ],
    out_specs=pl.BlockSpec((RB, G3D), lambda i: (i, 0)),
    out_shape=jax.ShapeDtypeStruct((MT, G3D), _f32),
)


def _l2_body(relu, agg_ref, inn_ref, onn_ref, w_ref, b_ref, g_ref):
    x = agg_ref[...] * inn_ref[...]
    y = jnp.dot(x, w_ref[...], preferred_element_type=_f32) + b_ref[...]
    if relu:
        y = jnp.maximum(y, 0.0) * onn_ref[...]
    g_ref[...] = y


def _make_l2(relu):
    return pl.pallas_call(
        functools.partial(_l2_body, relu),
        grid=(MT // RB,),
        in_specs=[
            pl.BlockSpec((RB, G3D), lambda i: (i, 0)),
            pl.BlockSpec((RB, 1), lambda i: (i, 0)),
            pl.BlockSpec((RB, 1), lambda i: (i, 0)),
            pl.BlockSpec((G3D, G3D), lambda i: (0, 0)),
            pl.BlockSpec((1, G3D), lambda i: (0, 0)),
        ],
        out_specs=pl.BlockSpec((RB, G3D), lambda i: (i, 0)),
        out_shape=jax.ShapeDtypeStruct((MT, G3D), _f32),
    )


_l2_call = _make_l2(True)
_l3_call = _make_l2(False)


def _row_mask(c):
    return (lax.broadcasted_iota(jnp.int32, (NC, G3D), 0) == c).astype(_f32)


def _cs_body(f_ref, s_ref):
    c = pl.program_id(0)
    i = pl.program_id(1)

    @pl.when(jnp.logical_and(c == 0, i == 0))
    def _():
        s_ref[...] = jnp.zeros_like(s_ref)

    part = jnp.sum(f_ref[...], axis=0, keepdims=True)
    s_ref[...] += _row_mask(c) * jnp.broadcast_to(part, (NC, G3D))


_cs_call = pl.pallas_call(
    _cs_body,
    grid=(NC, N // PB),
    in_specs=[pl.BlockSpec((PB, G3D), lambda c, i: (c * (N // PB) + i, 0))],
    out_specs=pl.BlockSpec((NC, G3D), lambda c, i: (0, 0)),
    out_shape=jax.ShapeDtypeStruct((NC, G3D), _f32),
)


def _p_body(f_ref, s_ref, aw_ref, p_ref):
    c = pl.program_id(0)
    i = pl.program_id(1)
    tg2 = jnp.tanh(jnp.dot(s_ref[...] / float(N), aw_ref[...],
                           preferred_element_type=_f32))
    tg = jnp.sum(_row_mask(c) * tg2, axis=0, keepdims=True)
    f = f_ref[...]
    logit = lax.dot_general(f, tg, (((1,), (1,)), ((), ())),
                            preferred_element_type=_f32)
    sig = 1.0 / (1.0 + jnp.exp(-logit))
    part = lax.dot_general(sig, f, (((0,), (0,)), ((), ())),
                           preferred_element_type=_f32)

    @pl.when(jnp.logical_and(c == 0, i == 0))
    def _():
        p_ref[...] = jnp.zeros_like(p_ref)

    p_ref[...] += _row_mask(c) * jnp.broadcast_to(part, (NC, G3D))


_p_call = pl.pallas_call(
    _p_body,
    grid=(NC, N // PB),
    in_specs=[
        pl.BlockSpec((PB, G3D), lambda c, i: (c * (N // PB) + i, 0)),
        pl.BlockSpec((NC, G3D), lambda c, i: (0, 0)),
        pl.BlockSpec((G3D, G3D), lambda c, i: (0, 0)),
    ],
    out_specs=pl.BlockSpec((NC, G3D), lambda c, i: (0, 0)),
    out_shape=jax.ShapeDtypeStruct((NC, G3D), _f32),
)


def kernel(edge_index_0, edge_index_1, W1, b1, W2, b2, W3, b3, att_W,
           tn_W, tn_Wb, tn_bias, fc1_W, fc1_b, fc2_W, fc2_b):
    i32 = jnp.int32
    pad_s = jnp.full((EPAD - E,), PAD_SRC, i32)
    pad_d = jnp.full((EPAD - E,), PAD_DST, i32)
    src_all = jnp.stack([
        jnp.concatenate([edge_index_0[0], pad_s]),
        jnp.concatenate([edge_index_1[0] + OFF1, pad_s]),
    ]).reshape(NC, ROWS, LW)
    dst_all = jnp.stack([
        jnp.concatenate([edge_index_0[1], pad_d]),
        jnp.concatenate([edge_index_1[1], pad_d]),
    ]).reshape(NC, ROWS, LW)

    in_deg, out_deg = _deg_kernel()(src_all, dst_all)
    inn, onn, g1 = _prep_call(in_deg.reshape(MT // LW, LW),
                              out_deg.reshape(MT // LW, LW))
    inn = inn.reshape(MT, 1)
    onn = onn.reshape(MT, 1)

    agg1 = _seg_kernel(1)(g1.reshape(MT), src_all, dst_all)
    g2 = _l1_call(agg1.reshape(MT, 1), inn, onn, W1, b1.reshape(1, G3D))
    agg2 = _seg_kernel(G3D)(g2, src_all, dst_all)
    g3 = _l2_call(agg2, inn, onn, W2, b2.reshape(1, G3D))
    agg3 = _seg_kernel(G3D)(g3, src_all, dst_all)
    f_full = _l3_call(agg3, inn, inn, W3, b3.reshape(1, G3D))
    f = jnp.concatenate([f_full[0:N], f_full[OFF1:OFF1 + N]], axis=0)
    colsum = _cs_call(f)
    P = _p_call(f, colsum, att_W)

    p0 = P[0].reshape(G3D, 1)
    p1 = P[1].reshape(G3D, 1)
    scoring = (p0.T @ tn_W.reshape(G3D, -1)).reshape(G3D, TN)
    scoring = scoring.T @ p1
    combined = jnp.concatenate([p0, p1], axis=0)
    block = tn_Wb @ combined
    scores = jax.nn.relu(scoring + block + tn_bias)
    s = scores.T
    s = jax.nn.relu(s @ fc1_W + fc1_b)
    return jax.nn.sigmoid(s @ fc2_W + fc2_b)


# speedup vs baseline: 25.0361x; 1.2104x over previous
import functools

import jax
import jax.numpy as jnp
from jax import lax
from jax.experimental import pallas as pl
from jax.experimental.pallas import tpu as pltpu
from jax.experimental.pallas import tpu_sc as plsc

N = 50_000
E = 1_600_000
G3D = 32
TN = 16
NC = 2
NS = 16
LW = 128
K = 8
RT = 784
NCHUNK = RT // K
ROWS = NS * RT
EPAD = ROWS * LW
ACC = 50_048
TPT = ACC // NS
MT = NC * ACC
OFF1 = ACC
PAD_SRC = OFF1 + N
PAD_DST = N
ZR = 136
RB = MT // 32
PB = 2000

_f32 = jnp.float32


@functools.lru_cache(maxsize=None)
def _mesh():
    return plsc.VectorSubcoreMesh(core_axis_name="c", subcore_axis_name="s",
                                  num_cores=NC, num_subcores=NS)


def _zero_1d(buf, n):
    def body(j, carry):
        buf[pl.ds(j * 16, 16)] = jnp.zeros((16,), _f32)
        return carry
    lax.fori_loop(0, n // 16, body, 0)


def _zero_2d(buf, n):
    def body(j, carry):
        buf[j, pl.ds(0, 16)] = jnp.zeros((16,), _f32)
        buf[j, pl.ds(16, 16)] = jnp.zeros((16,), _f32)
        return carry
    lax.fori_loop(0, n, body, 0)


def _deg_body(src_hbm, dst_hbm, in_deg, out_deg, didx0, didx1, sidx0, sidx1,
              ones_v, tbuf, in_acc, out_acc, sem0, sem1):
    c = lax.axis_index("c")
    s = lax.axis_index("s")
    didx = (didx0, didx1)
    sidx = (sidx0, sidx1)
    sem = (sem0, sem1)
    _zero_1d(tbuf, TPT)
    pltpu.sync_copy(tbuf, in_acc.at[pl.ds(s * TPT, TPT)])
    pltpu.sync_copy(tbuf, out_acc.at[pl.ds(s * TPT, TPT)])
    pltpu.sync_copy(tbuf, out_acc.at[pl.ds(ACC + s * TPT, TPT)])
    for i in range(LW // 16):
        ones_v[pl.ds(i * 16, 16)] = jnp.full((16,), 1.0, _f32)
    plsc.subcore_barrier()
    base = s * RT

    def fire(ci, b):
        r0 = base + ci * K
        pltpu.sync_copy(dst_hbm.at[c, pl.ds(r0, K)], didx[b])
        pltpu.sync_copy(src_hbm.at[c, pl.ds(r0, K)], sidx[b])
        for j in range(K):
            pltpu.async_copy(ones_v, in_acc.at[didx[b].at[j]], sem[b],
                             add=True)
        for j in range(K):
            pltpu.async_copy(ones_v, out_acc.at[sidx[b].at[j]], sem[b],
                             add=True)

    def drain(b):
        for j in range(K):
            pltpu.make_async_copy(ones_v, in_acc.at[didx[b].at[j]],
                                  sem[b]).wait()
        for j in range(K):
            pltpu.make_async_copy(ones_v, out_acc.at[sidx[b].at[j]],
                                  sem[b]).wait()

    fire(0, 0)
    fire(1, 1)

    def body(g, carry):
        for b in range(2):
            ci = 2 * g + b

            @pl.when(ci + 2 < NCHUNK)
            def _():
                drain(b)
                fire(ci + 2, b)
        return carry

    lax.fori_loop(0, NCHUNK // 2, body, 0)
    drain(0)
    drain(1)
    plsc.subcore_barrier()
    dst0 = c * ACC + s * TPT
    pltpu.sync_copy(in_acc.at[pl.ds(s * TPT, TPT)], tbuf)
    pltpu.sync_copy(tbuf, in_deg.at[pl.ds(dst0, TPT)])
    pltpu.sync_copy(out_acc.at[pl.ds(dst0, TPT)], tbuf)
    pltpu.sync_copy(tbuf, out_deg.at[pl.ds(dst0, TPT)])


@functools.lru_cache(maxsize=None)
def _deg_kernel():
    return pl.kernel(
        _deg_body,
        out_type=(jax.ShapeDtypeStruct((MT,), _f32),
                  jax.ShapeDtypeStruct((MT,), _f32)),
        mesh=_mesh(),
        scratch_types=[
            pltpu.VMEM((K, LW), jnp.int32),
            pltpu.VMEM((K, LW), jnp.int32),
            pltpu.VMEM((K, LW), jnp.int32),
            pltpu.VMEM((K, LW), jnp.int32),
            pltpu.VMEM((LW,), _f32),
            pltpu.VMEM((TPT,), _f32),
            pltpu.VMEM_SHARED((ACC,), _f32),
            pltpu.VMEM_SHARED((MT,), _f32),
            pltpu.SemaphoreType.DMA,
            pltpu.SemaphoreType.DMA,
        ],
        compiler_params=pltpu.CompilerParams(use_tc_tiling_on_sc=False),
    )


def _seg_body(D, g_hbm, src_hbm, dst_hbm, agg_out, sidx0, sidx1, didx0,
              didx1, rows0, rows1, tbuf, acc, gsem0, gsem1, ssem0, ssem1):
    KD = K if D == 1 else 2
    NCH = RT // KD
    c = lax.axis_index("c")
    s = lax.axis_index("s")
    sidx = (sidx0, sidx1)
    didx = (didx0, didx1)
    rows = (rows0, rows1)
    gsem = (gsem0, gsem1)
    ssem = (ssem0, ssem1)
    if D == 1:
        _zero_1d(tbuf, TPT)
        pltpu.sync_copy(tbuf, acc.at[pl.ds(s * TPT, TPT)])
    else:
        _zero_2d(tbuf, ZR)
        for q in range(TPT // ZR):
            pltpu.sync_copy(tbuf, acc.at[pl.ds(s * TPT + q * ZR, ZR)])
    plsc.subcore_barrier()
    base = s * RT

    def load_fire(ci, b):
        r0 = base + ci * KD
        pltpu.sync_copy(src_hbm.at[c, pl.ds(r0, KD)], sidx[b])
        pltpu.sync_copy(dst_hbm.at[c, pl.ds(r0, KD)], didx[b])
        for j in range(KD):
            pltpu.async_copy(g_hbm.at[sidx[b].at[j]], rows[b].at[j], gsem[b])

    def wait_gathers(b):
        for j in range(KD):
            pltpu.make_async_copy(g_hbm.at[sidx[b].at[j]], rows[b].at[j],
                                  gsem[b]).wait()

    def fire_scatters(b):
        for j in range(KD):
            pltpu.async_copy(rows[b].at[j], acc.at[didx[b].at[j]], ssem[b],
                             add=True)

    def wait_scatters(b):
        for j in range(KD):
            pltpu.make_async_copy(rows[b].at[j], acc.at[didx[b].at[j]],
                                  ssem[b]).wait()

    load_fire(0, 0)
    load_fire(1, 1)

    def body(g, carry):
        for b in range(2):
            ci = 2 * g + b
            wait_gathers(b)
            fire_scatters(b)

            @pl.when(ci + 2 < NCH)
            def _():
                wait_scatters(b)
                load_fire(ci + 2, b)
        return carry

    lax.fori_loop(0, NCH // 2, body, 0)
    wait_scatters(0)
    wait_scatters(1)
    plsc.subcore_barrier()
    if D == 1:
        dst0 = c * ACC + s * TPT
        pltpu.sync_copy(acc.at[pl.ds(s * TPT, TPT)], tbuf)
        pltpu.sync_copy(tbuf, agg_out.at[pl.ds(dst0, TPT)])
    else:
        for q in range(TPT // ZR):
            r0 = s * TPT + q * ZR
            pltpu.sync_copy(acc.at[pl.ds(r0, ZR)], tbuf)
            pltpu.sync_copy(tbuf, agg_out.at[pl.ds(c * ACC + r0, ZR)])


@functools.lru_cache(maxsize=None)
def _seg_kernel(D):
    if D == 1:
        KD = K
        gshape, rshape, tshape, ashape = (MT,), (KD, LW), (TPT,), (ACC,)
    else:
        KD = 2
        gshape, rshape, tshape, ashape = ((MT, D), (KD, LW, D), (ZR, D),
                                          (ACC, D))
    return pl.kernel(
        functools.partial(_seg_body, D),
        out_type=jax.ShapeDtypeStruct(gshape, _f32),
        mesh=_mesh(),
        scratch_types=[
            pltpu.VMEM((KD, LW), jnp.int32),
            pltpu.VMEM((KD, LW), jnp.int32),
            pltpu.VMEM((KD, LW), jnp.int32),
            pltpu.VMEM((KD, LW), jnp.int32),
            pltpu.VMEM(rshape, _f32),
            pltpu.VMEM(rshape, _f32),
            pltpu.VMEM(tshape, _f32),
            pltpu.VMEM_SHARED(ashape, _f32),
            pltpu.SemaphoreType.DMA,
            pltpu.SemaphoreType.DMA,
            pltpu.SemaphoreType.DMA,
            pltpu.SemaphoreType.DMA,
        ],
        compiler_params=pltpu.CompilerParams(use_tc_tiling_on_sc=False),
    )


def _prep_body(ind_ref, outd_ref, inn_ref, onn_ref, g1_ref):
    ind = ind_ref[...]
    onn = lax.rsqrt(jnp.maximum(outd_ref[...], 1.0))
    inn_ref[...] = lax.rsqrt(jnp.maximum(ind, 1.0))
    onn_ref[...] = onn
    g1_ref[...] = ind * onn


_prep_call = pl.pallas_call(
    _prep_body,
    out_shape=(jax.ShapeDtypeStruct((MT // LW, LW), _f32),) * 3,
)


def _l1_body(agg_ref, inn_ref, onn_ref, w_ref, b_ref, g_ref):
    x = agg_ref[...] * inn_ref[...]
    y = x * w_ref[...] + b_ref[...]
    g_ref[...] = jnp.maximum(y, 0.0) * onn_ref[...]


_l1_call = pl.pallas_call(
    _l1_body,
    grid=(MT // RB,),
    in_specs=[
        pl.BlockSpec((RB, 1), lambda i: (i, 0)),
        pl.BlockSpec((RB, 1), lambda i: (i, 0)),
        pl.BlockSpec((RB, 1), lambda i: (i, 0)),
        pl.BlockSpec((1, G3D), lambda i: (0, 0)),
        pl.BlockSpec((1, G3D), lambda i: (0, 0)),
    ],
    out_specs=pl.BlockSpec((RB, G3D), lambda i: (i, 0)),
    out_shape=jax.ShapeDtypeStruct((MT, G3D), _f32),
)


def _l2_body(relu, agg_ref, inn_ref, onn_ref, w_ref, b_ref, g_ref):
    x = agg_ref[...] * inn_ref[...]
    y = jnp.dot(x, w_ref[...], preferred_element_type=_f32) + b_ref[...]
    if relu:
        y = jnp.maximum(y, 0.0) * onn_ref[...]
    g_ref[...] = y


def _make_l2(relu):
    return pl.pallas_call(
        functools.partial(_l2_body, relu),
        grid=(MT // RB,),
        in_specs=[
            pl.BlockSpec((RB, G3D), lambda i: (i, 0)),
            pl.BlockSpec((RB, 1), lambda i: (i, 0)),
            pl.BlockSpec((RB, 1), lambda i: (i, 0)),
            pl.BlockSpec((G3D, G3D), lambda i: (0, 0)),
            pl.BlockSpec((1, G3D), lambda i: (0, 0)),
        ],
        out_specs=pl.BlockSpec((RB, G3D), lambda i: (i, 0)),
        out_shape=jax.ShapeDtypeStruct((MT, G3D), _f32),
    )


_l2_call = _make_l2(True)
_l3_call = _make_l2(False)


def _row_mask(c):
    return (lax.broadcasted_iota(jnp.int32, (NC, G3D), 0) == c).astype(_f32)


def _cs_body(f_ref, s_ref):
    c = pl.program_id(0)
    i = pl.program_id(1)

    @pl.when(jnp.logical_and(c == 0, i == 0))
    def _():
        s_ref[...] = jnp.zeros_like(s_ref)

    part = jnp.sum(f_ref[...], axis=0, keepdims=True)
    s_ref[...] += _row_mask(c) * jnp.broadcast_to(part, (NC, G3D))


_cs_call = pl.pallas_call(
    _cs_body,
    grid=(NC, N // PB),
    in_specs=[pl.BlockSpec((PB, G3D), lambda c, i: (c * (N // PB) + i, 0))],
    out_specs=pl.BlockSpec((NC, G3D), lambda c, i: (0, 0)),
    out_shape=jax.ShapeDtypeStruct((NC, G3D), _f32),
)


def _p_body(f_ref, s_ref, aw_ref, p_ref):
    c = pl.program_id(0)
    i = pl.program_id(1)
    tg2 = jnp.tanh(jnp.dot(s_ref[...] / float(N), aw_ref[...],
                           preferred_element_type=_f32))
    tg = jnp.sum(_row_mask(c) * tg2, axis=0, keepdims=True)
    f = f_ref[...]
    logit = lax.dot_general(f, tg, (((1,), (1,)), ((), ())),
                            preferred_element_type=_f32)
    sig = 1.0 / (1.0 + jnp.exp(-logit))
    part = lax.dot_general(sig, f, (((0,), (0,)), ((), ())),
                           preferred_element_type=_f32)

    @pl.when(jnp.logical_and(c == 0, i == 0))
    def _():
        p_ref[...] = jnp.zeros_like(p_ref)

    p_ref[...] += _row_mask(c) * jnp.broadcast_to(part, (NC, G3D))


_p_call = pl.pallas_call(
    _p_body,
    grid=(NC, N // PB),
    in_specs=[
        pl.BlockSpec((PB, G3D), lambda c, i: (c * (N // PB) + i, 0)),
        pl.BlockSpec((NC, G3D), lambda c, i: (0, 0)),
        pl.BlockSpec((G3D, G3D), lambda c, i: (0, 0)),
    ],
    out_specs=pl.BlockSpec((NC, G3D), lambda c, i: (0, 0)),
    out_shape=jax.ShapeDtypeStruct((NC, G3D), _f32),
)


def kernel(edge_index_0, edge_index_1, W1, b1, W2, b2, W3, b3, att_W,
           tn_W, tn_Wb, tn_bias, fc1_W, fc1_b, fc2_W, fc2_b):
    i32 = jnp.int32
    pad_s = jnp.full((EPAD - E,), PAD_SRC, i32)
    pad_d = jnp.full((EPAD - E,), PAD_DST, i32)
    src_all = jnp.stack([
        jnp.concatenate([edge_index_0[0], pad_s]),
        jnp.concatenate([edge_index_1[0] + OFF1, pad_s]),
    ]).reshape(NC, ROWS, LW)
    dst_all = jnp.stack([
        jnp.concatenate([edge_index_0[1], pad_d]),
        jnp.concatenate([edge_index_1[1], pad_d]),
    ]).reshape(NC, ROWS, LW)

    in_deg, out_deg = _deg_kernel()(src_all, dst_all)
    inn, onn, g1 = _prep_call(in_deg.reshape(MT // LW, LW),
                              out_deg.reshape(MT // LW, LW))
    inn = inn.reshape(MT, 1)
    onn = onn.reshape(MT, 1)

    agg1 = _seg_kernel(1)(g1.reshape(MT), src_all, dst_all)
    g2 = _l1_call(agg1.reshape(MT, 1), inn, onn, W1, b1.reshape(1, G3D))
    agg2 = _seg_kernel(G3D)(g2, src_all, dst_all)
    g3 = _l2_call(agg2, inn, onn, W2, b2.reshape(1, G3D))
    agg3 = _seg_kernel(G3D)(g3, src_all, dst_all)
    f_full = _l3_call(agg3, inn, inn, W3, b3.reshape(1, G3D))
    f = jnp.concatenate([f_full[0:N], f_full[OFF1:OFF1 + N]], axis=0)
    colsum = _cs_call(f)
    P = _p_call(f, colsum, att_W)

    p0 = P[0].reshape(G3D, 1)
    p1 = P[1].reshape(G3D, 1)
    scoring = (p0.T @ tn_W.reshape(G3D, -1)).reshape(G3D, TN)
    scoring = scoring.T @ p1
    combined = jnp.concatenate([p0, p1], axis=0)
    block = tn_Wb @ combined
    scores = jax.nn.relu(scoring + block + tn_bias)
    s = scores.T
    s = jax.nn.relu(s @ fc1_W + fc1_b)
    return jax.nn.sigmoid(s @ fc2_W + fc2_b)
